# baseline (speedup 1.0000x reference)
import jax
import jax.numpy as jnp
from jax import lax
from jax.experimental import pallas as pl
from jax.experimental.pallas import tpu as pltpu
from jax.experimental.pallas import tpu_sc as plsc

N = 10000
E = 160000
D = 256
H = 128
NC = 2
NS = 16
CHUNK = 64
NPH = 10
NBUF = 5
NFLY = 4
EPAD = 163840
WCH = EPAD // (NC * NS * CHUNK)
PCH = EPAD // (NS * CHUNK * NPH)
PAD_ROWS = 256
NROWS = N + PAD_ROWS
DEG_ROWS = 10496
DEG_TILE = DEG_ROWS // NS
DEG_OUT = 10112
ROW_TILE = 624


def _deg_body(dst3, deg_part, dstv, onesv, zerov, degsp):
    c = lax.axis_index("c")
    s = lax.axis_index("s")
    w = c * NS + s

    def _fill(ref, n16, val):
        def body(i, _):
            ref[pl.ds(i * 16, 16)] = jnp.full((16,), val, jnp.float32)
            return 0
        lax.fori_loop(0, n16, body, 0)

    _fill(zerov, DEG_TILE // 16, 0.0)
    _fill(onesv, CHUNK // 16, 1.0)
    pltpu.sync_copy(zerov, degsp.at[pl.ds(s * DEG_TILE, DEG_TILE)])
    plsc.subcore_barrier()

    pltpu.sync_copy(dst3.at[w], dstv)

    def edge_chunk(j, _):
        pltpu.sync_copy(onesv, degsp.at[dstv.at[j]], add=True)
        return 0
    lax.fori_loop(0, WCH, edge_chunk, 0)

    plsc.subcore_barrier()

    @pl.when(s == 0)
    def _():
        pltpu.sync_copy(degsp.at[pl.ds(0, DEG_OUT)], deg_part.at[c])


def _deg_kernel(dst3):
    mesh = plsc.VectorSubcoreMesh(core_axis_name="c", subcore_axis_name="s")
    f = pl.kernel(
        _deg_body,
        out_type=jax.ShapeDtypeStruct((NC, DEG_OUT), jnp.float32),
        mesh=mesh,
        scratch_types=[
            pltpu.VMEM((WCH, CHUNK), jnp.int32),
            pltpu.VMEM((CHUNK,), jnp.float32),
            pltpu.VMEM((DEG_TILE,), jnp.float32),
            pltpu.VMEM_SHARED((DEG_ROWS,), jnp.float32),
        ],
    )
    return f(dst3)


def _agg_body(src3, dst3, h0, h1, agg0, agg1, srcv, dstv, rowsv, aggsp, semg):
    c = lax.axis_index("c")
    s = lax.axis_index("s")

    def run(h_hbm, agg_hbm):
        @pl.when(s < NS - 1)
        def _():
            pltpu.sync_copy(h_hbm.at[pl.ds(s * ROW_TILE, ROW_TILE)],
                            aggsp.at[pl.ds(s * ROW_TILE, ROW_TILE)])

        @pl.when(s == NS - 1)
        def _():
            pltpu.sync_copy(h_hbm.at[pl.ds((NS - 1) * ROW_TILE, N - (NS - 1) * ROW_TILE)],
                            aggsp.at[pl.ds((NS - 1) * ROW_TILE, N - (NS - 1) * ROW_TILE)])

        pltpu.sync_copy(h_hbm.at[pl.ds(s * 16, 16)],
                        aggsp.at[pl.ds(N + s * 16, 16)])
        plsc.subcore_barrier()

        def phase(k, _):
            wrow = 2 * s + lax.div(k, NPH // 2)
            off = lax.rem(k, NPH // 2) * PCH
            pltpu.sync_copy(src3.at[wrow, pl.ds(off, PCH)], srcv)
            pltpu.sync_copy(dst3.at[wrow, pl.ds(off, PCH)], dstv)
            for b in range(NFLY):
                pltpu.async_copy(h_hbm.at[srcv.at[b]], rowsv.at[b], semg.at[b])

            def edge_chunk(j, _):
                jb = lax.rem(j, NBUF)
                pltpu.make_async_copy(h_hbm.at[srcv.at[j]], rowsv.at[jb],
                                      semg.at[jb]).wait()

                @pl.when(j + NFLY < PCH)
                def _():
                    nb = lax.rem(j + NFLY, NBUF)
                    pltpu.async_copy(h_hbm.at[srcv.at[j + NFLY]], rowsv.at[nb],
                                     semg.at[nb])

                pltpu.sync_copy(rowsv.at[jb], aggsp.at[dstv.at[j]], add=True)
                return 0
            lax.fori_loop(0, PCH, edge_chunk, 0)
            return 0
        lax.fori_loop(0, NPH, phase, 0)

        plsc.subcore_barrier()

        @pl.when(s < NS - 1)
        def _():
            pltpu.sync_copy(aggsp.at[pl.ds(s * ROW_TILE, ROW_TILE)],
                            agg_hbm.at[pl.ds(s * ROW_TILE, ROW_TILE)])

        @pl.when(s == NS - 1)
        def _():
            pltpu.sync_copy(aggsp.at[pl.ds((NS - 1) * ROW_TILE, N - (NS - 1) * ROW_TILE)],
                            agg_hbm.at[pl.ds((NS - 1) * ROW_TILE, N - (NS - 1) * ROW_TILE)])

    @pl.when(c == 0)
    def _():
        run(h0, agg0)

    @pl.when(c == 1)
    def _():
        run(h1, agg1)


def _agg_kernel(src3, dst3, h0, h1):
    mesh = plsc.VectorSubcoreMesh(core_axis_name="c", subcore_axis_name="s")
    f = pl.kernel(
        _agg_body,
        out_type=(jax.ShapeDtypeStruct((N, H), jnp.float32),
                  jax.ShapeDtypeStruct((N, H), jnp.float32)),
        mesh=mesh,
        scratch_types=[
            pltpu.VMEM((PCH, CHUNK), jnp.int32),
            pltpu.VMEM((PCH, CHUNK), jnp.int32),
            pltpu.VMEM((NBUF, CHUNK, H), jnp.float32),
            pltpu.VMEM_SHARED((NROWS, H), jnp.float32),
            pltpu.SemaphoreType.DMA((NBUF,)),
        ],
    )
    return f(src3, dst3, h0, h1)


def _tc_a_body(x_ref, w_ref, b_ref, degp_ref, h0_ref, h1_ref):
    dp = degp_ref[...]
    deg = dp[0, :N] + dp[1, :N] + 1.0
    dinv = lax.rsqrt(deg)
    h = jnp.dot(x_ref[...], w_ref[...], preferred_element_type=jnp.float32)
    h = (h + b_ref[...]) * dinv[:, None]
    h0_ref[...] = h[:, :H]
    h1_ref[...] = h[:, H:]


def _tc_a(x, W_conv, b_conv, deg_part):
    return pl.pallas_call(
        _tc_a_body,
        out_shape=[
            jax.ShapeDtypeStruct((N, H), jnp.float32),
            jax.ShapeDtypeStruct((N, H), jnp.float32),
        ],
    )(x, W_conv, b_conv.reshape(1, D), deg_part)


def _tc_b_body(a0_ref, a1_ref, degp_ref, w1_ref, b1_ref, w2_ref, b2_ref,
               emb_ref, log_ref):
    dp = degp_ref[...]
    deg = dp[0, :N] + dp[1, :N] + 1.0
    dinv = lax.rsqrt(deg)
    agg = jnp.concatenate([a0_ref[...], a1_ref[...]], axis=1) * dinv[:, None]
    emb_ref[...] = agg
    Xr = jnp.maximum(agg, 0.0)
    X = jnp.dot(Xr, w1_ref[...], preferred_element_type=jnp.float32) + b1_ref[...]
    X = jnp.dot(X, w2_ref[...], preferred_element_type=jnp.float32) + b2_ref[...]
    m = jnp.max(X, axis=1, keepdims=True)
    sh = X - m
    lse = jnp.log(jnp.sum(jnp.exp(sh), axis=1, keepdims=True))
    log_ref[...] = sh - lse


def _tc_b(agg0, agg1, deg_part, W1, b1, W2, b2):
    return pl.pallas_call(
        _tc_b_body,
        out_shape=[
            jax.ShapeDtypeStruct((N, D), jnp.float32),
            jax.ShapeDtypeStruct((N, D), jnp.float32),
        ],
    )(agg0, agg1, deg_part, W1, b1.reshape(1, D), W2, b2.reshape(1, D))


def kernel(x, edge_index, W_conv, b_conv, W1, b1, W2, b2):
    src = edge_index[0]
    dst = edge_index[1]
    npad = EPAD - E
    ar = jnp.arange(npad, dtype=jnp.int32)
    pad_src = (ar * 97) % N
    pad_dst = N + (ar % PAD_ROWS)
    src_p = jnp.concatenate([src, pad_src])
    dst_p = jnp.concatenate([dst, pad_dst])

    src3 = src_p.reshape(NC * NS, WCH, CHUNK)
    dst3 = dst_p.reshape(NC * NS, WCH, CHUNK)

    deg_part = _deg_kernel(dst3)
    h0, h1 = _tc_a(x, W_conv, b_conv, deg_part)
    agg0, agg1 = _agg_kernel(src3, dst3, h0, h1)
    embedding, logits = _tc_b(agg0, agg1, deg_part, W1, b1, W2, b2)
    return (embedding, logits)
